# baseline (speedup 1.0000x reference)
import jax
import jax.numpy as jnp
from jax import lax
from jax.experimental import pallas as pl
from jax.experimental.pallas import tpu as pltpu
from jax.experimental.pallas import tpu_sc as plsc

_NC = 2
_NS = 16
_NW = _NC * _NS
_CH = 16 * 1024


def _sc_body(x_hbm, pe_hbm, out_hbm, xb, peb, ob):
    total = x_hbm.shape[0]
    pe_total = pe_hbm.shape[0]
    wpw = total // _NW
    nch = wpw // _CH
    wid = lax.axis_index("s") * _NC + lax.axis_index("c")
    base = wid * wpw
    pbase = lax.rem(base, pe_total)

    def chunk(i, carry):
        off = base + i * _CH
        poff = pbase + i * _CH
        pltpu.sync_copy(x_hbm.at[pl.ds(off, _CH)], xb)
        pltpu.sync_copy(pe_hbm.at[pl.ds(poff, _CH)], peb)

        def sl(j, _):
            o = j * 16
            ob[pl.ds(o, 16)] = xb[pl.ds(o, 16)] + peb[pl.ds(o, 16)]
            return 0

        lax.fori_loop(0, _CH // 16, sl, 0, unroll=8)
        pltpu.sync_copy(ob, out_hbm.at[pl.ds(off, _CH)])
        return carry

    lax.fori_loop(0, nch, chunk, 0)


def kernel(x, pe):
    b, s, d = x.shape
    xf = x.reshape(b * s * d)
    pef = pe[:s].reshape(s * d)
    mesh = plsc.VectorSubcoreMesh(core_axis_name="c", subcore_axis_name="s")
    k = pl.kernel(
        _sc_body,
        mesh=mesh,
        out_type=jax.ShapeDtypeStruct((b * s * d,), x.dtype),
        scratch_types=[
            pltpu.VMEM((_CH,), jnp.float32),
            pltpu.VMEM((_CH,), jnp.float32),
            pltpu.VMEM((_CH,), jnp.float32),
        ],
    )
    return k(xf, pef).reshape(b, s, d)


# speedup vs baseline: 1.3410x; 1.3410x over previous
import jax
import jax.numpy as jnp
from jax import lax
from jax.experimental import pallas as pl
from jax.experimental.pallas import tpu as pltpu
from jax.experimental.pallas import tpu_sc as plsc

_NC = 2
_NS = 16
_NW = _NC * _NS
_CH = 16 * 1024


def _sc_body(x_hbm, pe_hbm, out_hbm,
             xb0, xb1, pb0, pb1, ob0, ob1,
             sx0, sx1, sp0, sp1, so0, so1):
    xb = (xb0, xb1)
    pb = (pb0, pb1)
    ob = (ob0, ob1)
    sx = (sx0, sx1)
    sp = (sp0, sp1)
    so = (so0, so1)

    total = x_hbm.shape[0]
    pe_total = pe_hbm.shape[0]
    wpw = total // _NW
    nch = wpw // _CH
    wid = lax.axis_index("s") * _NC + lax.axis_index("c")
    base = wid * wpw
    pbase = lax.rem(base, pe_total)

    def start_in(i, b):
        pltpu.make_async_copy(
            x_hbm.at[pl.ds(base + i * _CH, _CH)], xb[b], sx[b]).start()
        pltpu.make_async_copy(
            pe_hbm.at[pl.ds(pbase + i * _CH, _CH)], pb[b], sp[b]).start()

    start_in(0, 0)
    start_in(1, 1)

    def outer(g, carry):
        for b in range(2):
            i = g * 2 + b
            pltpu.make_async_copy(
                x_hbm.at[pl.ds(base + i * _CH, _CH)], xb[b], sx[b]).wait()
            pltpu.make_async_copy(
                pe_hbm.at[pl.ds(pbase + i * _CH, _CH)], pb[b], sp[b]).wait()

            @pl.when(i >= 2)
            def _():
                pltpu.make_async_copy(
                    ob[b], out_hbm.at[pl.ds(base + (i - 2) * _CH, _CH)],
                    so[b]).wait()

            def sl(j, _):
                o = j * 16
                ob[b][pl.ds(o, 16)] = xb[b][pl.ds(o, 16)] + pb[b][pl.ds(o, 16)]
                return 0

            lax.fori_loop(0, _CH // 16, sl, 0, unroll=8)

            pltpu.make_async_copy(
                ob[b], out_hbm.at[pl.ds(base + i * _CH, _CH)], so[b]).start()

            @pl.when(i + 2 < nch)
            def _():
                start_in(i + 2, b)
        return carry

    lax.fori_loop(0, nch // 2, outer, 0)

    pltpu.make_async_copy(
        ob[0], out_hbm.at[pl.ds(base + (nch - 2) * _CH, _CH)], so[0]).wait()
    pltpu.make_async_copy(
        ob[1], out_hbm.at[pl.ds(base + (nch - 1) * _CH, _CH)], so[1]).wait()


def kernel(x, pe):
    b, s, d = x.shape
    xf = x.reshape(b * s * d)
    pef = pe[:s].reshape(s * d)
    mesh = plsc.VectorSubcoreMesh(core_axis_name="c", subcore_axis_name="s")
    k = pl.kernel(
        _sc_body,
        mesh=mesh,
        out_type=jax.ShapeDtypeStruct((b * s * d,), x.dtype),
        scratch_types=(
            [pltpu.VMEM((_CH,), jnp.float32) for _ in range(6)]
            + [pltpu.SemaphoreType.DMA for _ in range(6)]
        ),
    )
    return k(xf, pef).reshape(b, s, d)


# speedup vs baseline: 1.9850x; 1.4802x over previous
import jax
import jax.numpy as jnp
from jax import lax
from jax.experimental import pallas as pl
from jax.experimental.pallas import tpu as pltpu
from jax.experimental.pallas import tpu_sc as plsc

_NC = 2
_NS = 16
_NW = _NC * _NS
_CH = 16 * 1024


def _sc_body(x_hbm, pe_hbm, out_hbm,
             xb0, xb1, pb0, pb1, ob0, ob1,
             sx0, sx1, sp0, sp1, so0, so1):
    xb = (xb0, xb1)
    pb = (pb0, pb1)
    ob = (ob0, ob1)
    sx = (sx0, sx1)
    sp = (sp0, sp1)
    so = (so0, so1)

    total = x_hbm.shape[0]
    pe_total = pe_hbm.shape[0]
    wpw = total // _NW
    nch = wpw // _CH
    wid = lax.axis_index("s") * _NC + lax.axis_index("c")
    base = wid * wpw
    pbase = lax.rem(base, pe_total)

    def start_in(i, b):
        pltpu.make_async_copy(
            x_hbm.at[pl.ds(base + i * _CH, _CH)], xb[b], sx[b]).start()
        pltpu.make_async_copy(
            pe_hbm.at[pl.ds(pbase + i * _CH, _CH)], pb[b], sp[b]).start()

    start_in(0, 0)
    start_in(1, 1)

    def outer(g, carry):
        for b in range(2):
            i = g * 2 + b
            pltpu.make_async_copy(
                x_hbm.at[pl.ds(base + i * _CH, _CH)], xb[b], sx[b]).wait()
            pltpu.make_async_copy(
                pe_hbm.at[pl.ds(pbase + i * _CH, _CH)], pb[b], sp[b]).wait()

            @pl.when(i >= 2)
            def _():
                pltpu.make_async_copy(
                    ob[b], out_hbm.at[pl.ds(base + (i - 2) * _CH, _CH)],
                    so[b]).wait()

            @plsc.parallel_loop(0, _CH // 16, 1, unroll=8)
            def _(j):
                o = j * 16
                ob[b][pl.ds(o, 16)] = xb[b][pl.ds(o, 16)] + pb[b][pl.ds(o, 16)]

            pltpu.make_async_copy(
                ob[b], out_hbm.at[pl.ds(base + i * _CH, _CH)], so[b]).start()

            @pl.when(i + 2 < nch)
            def _():
                start_in(i + 2, b)
        return carry

    lax.fori_loop(0, nch // 2, outer, 0)

    pltpu.make_async_copy(
        ob[0], out_hbm.at[pl.ds(base + (nch - 2) * _CH, _CH)], so[0]).wait()
    pltpu.make_async_copy(
        ob[1], out_hbm.at[pl.ds(base + (nch - 1) * _CH, _CH)], so[1]).wait()


def kernel(x, pe):
    b, s, d = x.shape
    xf = x.reshape(b * s * d)
    pef = pe[:s].reshape(s * d)
    mesh = plsc.VectorSubcoreMesh(core_axis_name="c", subcore_axis_name="s")
    k = pl.kernel(
        _sc_body,
        mesh=mesh,
        out_type=jax.ShapeDtypeStruct((b * s * d,), x.dtype),
        scratch_types=(
            [pltpu.VMEM((_CH,), jnp.float32) for _ in range(6)]
            + [pltpu.SemaphoreType.DMA for _ in range(6)]
        ),
    )
    return k(xf, pef).reshape(b, s, d)


# speedup vs baseline: 1.9859x; 1.0004x over previous
import jax
import jax.numpy as jnp
from jax import lax
from jax.experimental import pallas as pl
from jax.experimental.pallas import tpu as pltpu
from jax.experimental.pallas import tpu_sc as plsc

_NC = 2
_NS = 16
_NW = _NC * _NS
_CH = 16 * 1024


def _sc_body(x_hbm, pe_hbm, out_hbm,
             xb0, xb1, pb0, pb1, ob0, ob1,
             sx0, sx1, sp0, sp1, so0, so1):
    xb = (xb0, xb1)
    pb = (pb0, pb1)
    ob = (ob0, ob1)
    sx = (sx0, sx1)
    sp = (sp0, sp1)
    so = (so0, so1)

    total = x_hbm.shape[0]
    pe_total = pe_hbm.shape[0]
    wpw = total // _NW
    nch = wpw // _CH
    wid = lax.axis_index("s") * _NC + lax.axis_index("c")
    base = wid * wpw
    pbase = lax.rem(base, pe_total)

    def start_in(i, b):
        pltpu.make_async_copy(
            x_hbm.at[pl.ds(base + i * _CH, _CH)], xb[b], sx[b]).start()
        pltpu.make_async_copy(
            pe_hbm.at[pl.ds(pbase + i * _CH, _CH)], pb[b], sp[b]).start()

    start_in(0, 0)
    start_in(1, 1)

    def outer(g, carry):
        for b in range(2):
            i = g * 2 + b
            pltpu.make_async_copy(
                x_hbm.at[pl.ds(base + i * _CH, _CH)], xb[b], sx[b]).wait()
            pltpu.make_async_copy(
                pe_hbm.at[pl.ds(pbase + i * _CH, _CH)], pb[b], sp[b]).wait()

            @pl.when(i >= 2)
            def _():
                pltpu.make_async_copy(
                    ob[b], out_hbm.at[pl.ds(base + (i - 2) * _CH, _CH)],
                    so[b]).wait()

            @plsc.parallel_loop(0, _CH // 16, 1, unroll=32)
            def _(j):
                o = j * 16
                ob[b][pl.ds(o, 16)] = xb[b][pl.ds(o, 16)] + pb[b][pl.ds(o, 16)]

            pltpu.make_async_copy(
                ob[b], out_hbm.at[pl.ds(base + i * _CH, _CH)], so[b]).start()

            @pl.when(i + 2 < nch)
            def _():
                start_in(i + 2, b)
        return carry

    lax.fori_loop(0, nch // 2, outer, 0)

    pltpu.make_async_copy(
        ob[0], out_hbm.at[pl.ds(base + (nch - 2) * _CH, _CH)], so[0]).wait()
    pltpu.make_async_copy(
        ob[1], out_hbm.at[pl.ds(base + (nch - 1) * _CH, _CH)], so[1]).wait()


def kernel(x, pe):
    b, s, d = x.shape
    xf = x.reshape(b * s * d)
    pef = pe[:s].reshape(s * d)
    mesh = plsc.VectorSubcoreMesh(core_axis_name="c", subcore_axis_name="s")
    k = pl.kernel(
        _sc_body,
        mesh=mesh,
        out_type=jax.ShapeDtypeStruct((b * s * d,), x.dtype),
        scratch_types=(
            [pltpu.VMEM((_CH,), jnp.float32) for _ in range(6)]
            + [pltpu.SemaphoreType.DMA for _ in range(6)]
        ),
    )
    return k(xf, pef).reshape(b, s, d)


# speedup vs baseline: 9.0511x; 4.5578x over previous
import jax
import jax.numpy as jnp
from jax.experimental import pallas as pl
from jax.experimental.pallas import tpu as pltpu

_BS = 2048


def _add_kernel(x_ref, pe_ref, o_ref):
    o_ref[...] = x_ref[...] + pe_ref[...]


def kernel(x, pe):
    b, s, d = x.shape
    nsb = s // _BS
    return pl.pallas_call(
        _add_kernel,
        grid=(nsb, b),
        in_specs=[
            pl.BlockSpec((1, _BS, d), lambda i, j: (j, i, 0)),
            pl.BlockSpec((_BS, d), lambda i, j: (i, 0)),
        ],
        out_specs=pl.BlockSpec((1, _BS, d), lambda i, j: (j, i, 0)),
        out_shape=jax.ShapeDtypeStruct((b, s, d), x.dtype),
        compiler_params=pltpu.CompilerParams(
            dimension_semantics=("parallel", "parallel"),
        ),
    )(x, pe[:s])
